# baseline (speedup 1.0000x reference)
import functools

import jax
import jax.numpy as jnp
from jax import lax
from jax.experimental import pallas as pl
from jax.experimental.pallas import tpu as pltpu
from jax.experimental.pallas import tpu_sc as plsc

_NBUF = 4
_IBLK = 4


def _transpose_block(r, lane, stages=(1, 2, 4, 8)):
    t = list(r)
    for s in stages:
        idx_m = (lane - s) & 15
        idx_p = (lane + s) & 15
        m = (lane & s) == 0
        for i in range(16):
            if i & s:
                continue
            x, y = t[i], t[i + s]
            z = y.at[idx_m].get(mode="promise_in_bounds")
            w = x.at[idx_p].get(mode="promise_in_bounds")
            t[i] = jnp.where(m, x, z)
            t[i + s] = jnp.where(m, w, y)
    return t


def _make_transpose(V, D):
    info = plsc.get_sparse_core_info()
    NC, NS = info.num_cores, info.num_subcores
    NW = NC * NS
    NVG = V // 128
    per_w = NVG // NW
    extras = NVG - per_w * NW
    n_main = per_w - 2
    assert n_main % 2 == 0

    mesh = plsc.VectorSubcoreMesh(core_axis_name="c", subcore_axis_name="s")

    @functools.partial(
        pl.kernel,
        out_type=jax.ShapeDtypeStruct((V // 2, 128), jnp.float32),
        mesh=mesh,
        scratch_types=[
            pltpu.VMEM((64, 129), jnp.float32),
            pltpu.VMEM((64, 129), jnp.float32),
            pltpu.VMEM((64, 128), jnp.float32),
            pltpu.VMEM((64, 128), jnp.float32),
            pltpu.VMEM((32, 128), jnp.float32),
            pltpu.SemaphoreType.DMA,
            pltpu.SemaphoreType.DMA,
            pltpu.SemaphoreType.DMA,
            pltpu.SemaphoreType.DMA,
        ],
        compiler_params=pltpu.CompilerParams(needs_layout_passes=False),
    )
    def transpose_kernel(wt_hbm, wtail_hbm, wpk_hbm, in0, in1, out0, out1,
                         tailbuf, is0, is1, os0, os1):
        wid = lax.axis_index("s") * NC + lax.axis_index("c")
        vg0 = wid * per_w
        inb, outb = [in0, in1], [out0, out1]
        isem, osem = [is0, is1], [os0, os1]
        lane = lax.iota(jnp.int32, 16)

        def i_copy(k, b):
            return pltpu.make_async_copy(
                wt_hbm.at[:, pl.ds((vg0 + k) * 128, 128)],
                inb[b].at[:, pl.ds(0, 128)], isem[b])

        def o_copy(k, b):
            return pltpu.make_async_copy(
                outb[b], wpk_hbm.at[pl.ds((vg0 + k) * 64, 64)], osem[b])

        def shuffle_refs(src, dst):
            m1 = (lane & 1) == 0

            def bbody(db, carry):
                r0 = db * 16
                for vb in range(8):
                    c0 = vb * 16
                    if vb == 0:
                        r = [src[r0 + i, pl.ds(c0, 16)] for i in range(16)]
                        t = _transpose_block(r, lane)
                    else:
                        t = []
                        for i in range(0, 16, 2):
                            xx = src[r0 + i, pl.ds(c0, 16)]
                            yy = src[r0 + i + 1, pl.ds(c0, 16)]
                            zz = src[r0 + i + 1, pl.ds(c0 - 1, 16)]
                            ww = src[r0 + i, pl.ds(c0 + 1, 16)]
                            t.append(jnp.where(m1, xx, zz))
                            t.append(jnp.where(m1, ww, yy))
                        t = _transpose_block(t, lane, stages=(2, 4, 8))
                    for j in range(16):
                        dst[vb * 8 + (j >> 1),
                            pl.ds((j & 1) * 64 + r0, 16)] = t[j]
                return carry
            lax.fori_loop(0, 4, bbody, 0)

        i_copy(0, 0).start()
        i_copy(1, 1).start()

        def body(p, carry):
            for j in range(2):
                k = 2 * p + j
                i_copy(k, j).wait()

                @pl.when(k >= 2)
                def _():
                    o_copy(k - 2, j).wait()

                shuffle_refs(inb[j], outb[j])
                o_copy(k, j).start()
                i_copy(k + 2, j).start()
            return carry

        lax.fori_loop(0, n_main // 2, body, 0)

        for k in (per_w - 2, per_w - 1):
            j = k & 1
            i_copy(k, j).wait()
            o_copy(k - 2, j).wait()
            shuffle_refs(inb[j], outb[j])
            o_copy(k, j).start()
        o_copy(per_w - 2, 0).wait()
        o_copy(per_w - 1, 1).wait()

        @pl.when(wid < extras)
        def _():
            evg = NW * per_w + wid
            pltpu.make_async_copy(
                wt_hbm.at[:, pl.ds(evg * 128, 128)],
                in0.at[:, pl.ds(0, 128)], is0).start()
            pltpu.make_async_copy(
                wt_hbm.at[:, pl.ds(evg * 128, 128)],
                in0.at[:, pl.ds(0, 128)], is0).wait()
            shuffle_refs(in0, out0)
            pltpu.make_async_copy(
                out0, wpk_hbm.at[pl.ds(evg * 64, 64)], os0).start()
            pltpu.make_async_copy(
                out0, wpk_hbm.at[pl.ds(evg * 64, 64)], os0).wait()

        @pl.when(wid == extras)
        def _():
            pltpu.sync_copy(wtail_hbm, tailbuf)
            pltpu.sync_copy(tailbuf, wpk_hbm.at[pl.ds(NVG * 64, 32)])

    return transpose_kernel


def _make_gather(V, D, NI, S, S56):
    info = plsc.get_sparse_core_info()
    NC, NS = info.num_cores, info.num_subcores
    NW = NC * NS
    B = NI * S
    assert B % NW == 0 and NI % NW == 0
    b_per_w = B // NW
    i_per_w = NI // NW
    chunk = _IBLK * S
    assert b_per_w % chunk == 0
    n_chunks = b_per_w // chunk
    assert n_chunks >= _NBUF and (n_chunks - 4) % _NBUF == 0

    mesh = plsc.VectorSubcoreMesh(core_axis_name="c", subcore_axis_name="s")

    scratch = [pltpu.VMEM((b_per_w,), jnp.int32)]
    scratch += [pltpu.VMEM((chunk, D), jnp.float32) for _ in range(_NBUF)]
    scratch += [pltpu.SemaphoreType.DMA for _ in range(2 * _NBUF)]

    @functools.partial(
        pl.kernel,
        out_type=jax.ShapeDtypeStruct((NI * S56, 2 * D), jnp.float32),
        mesh=mesh,
        scratch_types=scratch,
        compiler_params=pltpu.CompilerParams(use_tc_tiling_on_sc=False),
    )
    def gather_kernel(idx_hbm, table_hbm, out_hbm, idx_all, *bufs):
        rows = bufs[:_NBUF]
        gsem = bufs[_NBUF:2 * _NBUF]
        wsem = bufs[2 * _NBUF:]
        wid = lax.axis_index("s") * NC + lax.axis_index("c")
        base = wid * b_per_w
        obase = wid * i_per_w * S56

        pltpu.sync_copy(idx_hbm.at[pl.ds(base, b_per_w)], idx_all)

        def g_copy(c, b):
            return pltpu.make_async_copy(
                table_hbm.at[idx_all.at[pl.ds(c * chunk, chunk)]],
                rows[b], gsem[b])

        def w_copy_start(c, b):
            for j in range(_IBLK):
                pltpu.make_async_copy(
                    rows[b].at[pl.ds(j * S, S)],
                    out_hbm.at[pl.ds(obase + (c * _IBLK + j) * S56, S),
                               pl.ds(0, D)],
                    wsem[b]).start()

        def w_copy_wait(c, b):
            for j in range(_IBLK):
                pltpu.make_async_copy(
                    rows[b].at[pl.ds(j * S, S)],
                    out_hbm.at[pl.ds(obase + (c * _IBLK + j) * S56, S),
                               pl.ds(0, D)],
                    wsem[b]).wait()

        g_copy(0, 0).start()
        g_copy(1, 1).start()
        g_copy(0, 0).wait()
        w_copy_start(0, 0)
        g_copy(2, 2).start()
        g_copy(1, 1).wait()
        w_copy_start(1, 1)
        g_copy(3, 3).start()

        def body(p, carry):
            c0 = 2 + p * _NBUF
            for j in range(_NBUF):
                c = c0 + j
                b = (2 + j) % _NBUF
                b2 = j
                g_copy(c, b).wait()
                w_copy_start(c, b)
                w_copy_wait(c - 2, b2)
                g_copy(c + 2, b2).start()
            return carry

        n_main = (n_chunks - 4) // _NBUF
        lax.fori_loop(0, n_main, body, 0)

        cA, cB = n_chunks - 2, n_chunks - 1
        bA, bB = cA % _NBUF, cB % _NBUF
        g_copy(cA, bA).wait()
        w_copy_start(cA, bA)
        g_copy(cB, bB).wait()
        w_copy_start(cB, bB)
        w_copy_wait(n_chunks - 4, (n_chunks - 4) % _NBUF)
        w_copy_wait(n_chunks - 3, (n_chunks - 3) % _NBUF)
        w_copy_wait(cA, bA)
        w_copy_wait(cB, bB)

    return gather_kernel


def kernel(x, weight):
    V, D = weight.shape
    NI, S = x.shape
    S56 = (S + 7) // 8 * 8
    wT = weight.T
    wtail = lax.slice(weight, (V - 64, 0), (V, D)).reshape(32, 128)
    wpk = _make_transpose(V, D)(wT, wtail)
    w64 = wpk.reshape(V, D)
    xf = x.reshape(NI * S).astype(jnp.int32)
    out56 = _make_gather(V, D, NI, S, S56)(xf, w64)
    out3 = out56.reshape(NI, S56, 2 * D)
    return lax.slice(out3, (0, 0, 0), (NI, S, D))


# speedup vs baseline: 1.0431x; 1.0431x over previous
import functools

import jax
import jax.numpy as jnp
from jax import lax
from jax.experimental import pallas as pl
from jax.experimental.pallas import tpu as pltpu
from jax.experimental.pallas import tpu_sc as plsc

_NBUF = 4
_IBLK = 4


def _transpose_block(r, lane):
    t = list(r)
    for s in (1, 2, 4, 8):
        idx_m = (lane - s) & 15
        idx_p = (lane + s) & 15
        m = (lane & s) == 0
        for i in range(16):
            if i & s:
                continue
            x, y = t[i], t[i + s]
            z = y.at[idx_m].get(mode="promise_in_bounds")
            w = x.at[idx_p].get(mode="promise_in_bounds")
            t[i] = jnp.where(m, x, z)
            t[i + s] = jnp.where(m, w, y)
    return t


def _make_transpose(V, D):
    info = plsc.get_sparse_core_info()
    NC, NS = info.num_cores, info.num_subcores
    NW = NC * NS
    NVG = V // 128
    per_w = NVG // NW
    extras = NVG - per_w * NW
    n_main = per_w - 2
    assert n_main % 2 == 0

    mesh = plsc.VectorSubcoreMesh(core_axis_name="c", subcore_axis_name="s")

    @functools.partial(
        pl.kernel,
        out_type=jax.ShapeDtypeStruct((V // 2, 128), jnp.float32),
        mesh=mesh,
        scratch_types=[
            pltpu.VMEM((64, 128), jnp.float32),
            pltpu.VMEM((64, 128), jnp.float32),
            pltpu.VMEM((64, 128), jnp.float32),
            pltpu.VMEM((64, 128), jnp.float32),
            pltpu.VMEM((32, 128), jnp.float32),
            pltpu.SemaphoreType.DMA,
            pltpu.SemaphoreType.DMA,
            pltpu.SemaphoreType.DMA,
            pltpu.SemaphoreType.DMA,
        ],
        compiler_params=pltpu.CompilerParams(needs_layout_passes=False),
    )
    def transpose_kernel(wt_hbm, wtail_hbm, wpk_hbm, in0, in1, out0, out1,
                         tailbuf, is0, is1, os0, os1):
        wid = lax.axis_index("s") * NC + lax.axis_index("c")
        vg0 = wid * per_w
        inb, outb = [in0, in1], [out0, out1]
        isem, osem = [is0, is1], [os0, os1]
        lane = lax.iota(jnp.int32, 16)

        def i_copy(k, b):
            return pltpu.make_async_copy(
                wt_hbm.at[:, pl.ds((vg0 + k) * 128, 128)], inb[b], isem[b])

        def o_copy(k, b):
            return pltpu.make_async_copy(
                outb[b], wpk_hbm.at[pl.ds((vg0 + k) * 64, 64)], osem[b])

        def shuffle_refs(src, dst):
            def bbody(bk, carry):
                db = bk >> 3
                vb = bk & 7
                r0 = db * 16
                c0 = vb * 16
                r = [src[r0 + i, pl.ds(c0, 16)] for i in range(16)]
                t = _transpose_block(r, lane)
                for j in range(16):
                    dst[vb * 8 + (j >> 1),
                        pl.ds((j & 1) * 64 + r0, 16)] = t[j]
                return carry
            lax.fori_loop(0, 32, bbody, 0)

        i_copy(0, 0).start()
        i_copy(1, 1).start()

        def body(p, carry):
            for j in range(2):
                k = 2 * p + j
                i_copy(k, j).wait()

                @pl.when(k >= 2)
                def _():
                    o_copy(k - 2, j).wait()

                shuffle_refs(inb[j], outb[j])
                o_copy(k, j).start()
                i_copy(k + 2, j).start()
            return carry

        lax.fori_loop(0, n_main // 2, body, 0)

        for k in (per_w - 2, per_w - 1):
            j = k & 1
            i_copy(k, j).wait()
            o_copy(k - 2, j).wait()
            shuffle_refs(inb[j], outb[j])
            o_copy(k, j).start()
        o_copy(per_w - 2, 0).wait()
        o_copy(per_w - 1, 1).wait()

        @pl.when(wid < extras)
        def _():
            evg = NW * per_w + wid
            pltpu.make_async_copy(
                wt_hbm.at[:, pl.ds(evg * 128, 128)], in0, is0).start()
            pltpu.make_async_copy(
                wt_hbm.at[:, pl.ds(evg * 128, 128)], in0, is0).wait()
            shuffle_refs(in0, out0)
            pltpu.make_async_copy(
                out0, wpk_hbm.at[pl.ds(evg * 64, 64)], os0).start()
            pltpu.make_async_copy(
                out0, wpk_hbm.at[pl.ds(evg * 64, 64)], os0).wait()

        @pl.when(wid == extras)
        def _():
            pltpu.sync_copy(wtail_hbm, tailbuf)
            pltpu.sync_copy(tailbuf, wpk_hbm.at[pl.ds(NVG * 64, 32)])

    return transpose_kernel


def _make_gather(V, D, NI, S, S56):
    info = plsc.get_sparse_core_info()
    NC, NS = info.num_cores, info.num_subcores
    NW = NC * NS
    B = NI * S
    assert B % NW == 0 and NI % NW == 0
    b_per_w = B // NW
    i_per_w = NI // NW
    chunk = _IBLK * S
    assert b_per_w % chunk == 0
    n_chunks = b_per_w // chunk
    assert n_chunks >= _NBUF and (n_chunks - 4) % _NBUF == 0

    mesh = plsc.VectorSubcoreMesh(core_axis_name="c", subcore_axis_name="s")

    scratch = [pltpu.VMEM((b_per_w,), jnp.int32)]
    scratch += [pltpu.VMEM((chunk, D), jnp.float32) for _ in range(_NBUF)]
    scratch += [pltpu.SemaphoreType.DMA for _ in range(2 * _NBUF)]

    @functools.partial(
        pl.kernel,
        out_type=jax.ShapeDtypeStruct((NI * S56, 2 * D), jnp.float32),
        mesh=mesh,
        scratch_types=scratch,
        compiler_params=pltpu.CompilerParams(use_tc_tiling_on_sc=False),
    )
    def gather_kernel(idx_hbm, table_hbm, out_hbm, idx_all, *bufs):
        rows = bufs[:_NBUF]
        gsem = bufs[_NBUF:2 * _NBUF]
        wsem = bufs[2 * _NBUF:]
        wid = lax.axis_index("s") * NC + lax.axis_index("c")
        base = wid * b_per_w
        obase = wid * i_per_w * S56

        pltpu.sync_copy(idx_hbm.at[pl.ds(base, b_per_w)], idx_all)

        def g_copy(c, b):
            return pltpu.make_async_copy(
                table_hbm.at[idx_all.at[pl.ds(c * chunk, chunk)]],
                rows[b], gsem[b])

        def w_copy_start(c, b):
            for j in range(_IBLK):
                pltpu.make_async_copy(
                    rows[b].at[pl.ds(j * S, S)],
                    out_hbm.at[pl.ds(obase + (c * _IBLK + j) * S56, S),
                               pl.ds(0, D)],
                    wsem[b]).start()

        def w_copy_wait(c, b):
            for j in range(_IBLK):
                pltpu.make_async_copy(
                    rows[b].at[pl.ds(j * S, S)],
                    out_hbm.at[pl.ds(obase + (c * _IBLK + j) * S56, S),
                               pl.ds(0, D)],
                    wsem[b]).wait()

        g_copy(0, 0).start()
        g_copy(1, 1).start()
        g_copy(0, 0).wait()
        w_copy_start(0, 0)
        g_copy(2, 2).start()
        g_copy(1, 1).wait()
        w_copy_start(1, 1)
        g_copy(3, 3).start()

        def body(p, carry):
            c0 = 2 + p * _NBUF
            for j in range(_NBUF):
                c = c0 + j
                b = (2 + j) % _NBUF
                b2 = j
                g_copy(c, b).wait()
                w_copy_start(c, b)
                w_copy_wait(c - 2, b2)
                g_copy(c + 2, b2).start()
            return carry

        n_main = (n_chunks - 4) // _NBUF
        lax.fori_loop(0, n_main, body, 0)

        cA, cB = n_chunks - 2, n_chunks - 1
        bA, bB = cA % _NBUF, cB % _NBUF
        g_copy(cA, bA).wait()
        w_copy_start(cA, bA)
        g_copy(cB, bB).wait()
        w_copy_start(cB, bB)
        w_copy_wait(n_chunks - 4, (n_chunks - 4) % _NBUF)
        w_copy_wait(n_chunks - 3, (n_chunks - 3) % _NBUF)
        w_copy_wait(cA, bA)
        w_copy_wait(cB, bB)

    return gather_kernel


def kernel(x, weight):
    V, D = weight.shape
    NI, S = x.shape
    S56 = (S + 7) // 8 * 8
    wT = weight.T
    wtail = lax.slice(weight, (V - 64, 0), (V, D)).reshape(32, 128)
    wpk = _make_transpose(V, D)(wT, wtail)
    w64 = wpk.reshape(V, D)
    xf = x.reshape(NI * S).astype(jnp.int32)
    out56 = _make_gather(V, D, NI, S, S56)(xf, w64)
    out3 = out56.reshape(NI, S56, 2 * D)
    return lax.slice(out3, (0, 0, 0), (NI, S, D))
